# baseline (speedup 1.0000x reference)
import jax
import jax.numpy as jnp
from jax import lax
from jax.experimental import pallas as pl
from jax.experimental.pallas import tpu as pltpu
from jax.experimental.pallas import tpu_sc as plsc

F32 = jnp.float32

_NT = 16
_LANES = 16
_WS = 16
_CE = 1024
_SUB = 128
_RCH = 512


def _splat(v16, e):
    return lax.gather(
        v16, jnp.full((_LANES, 1), e, jnp.int32),
        dimension_numbers=lax.GatherDimensionNumbers(
            offset_dims=(), collapsed_slice_dims=(0,), start_index_map=(0,)),
        slice_sizes=(1,),
        mode=lax.GatherScatterMode.PROMISE_IN_BOUNDS)


def _sc_spmm(htable, xtable, src_flat, dst2, ew_flat, zrows,
             num_hslabs, n_nodes, n_pad):
    e_pad = src_flat.shape[0]
    te = e_pad // _NT
    nch = te // _CE
    nrch = n_pad // _RCH
    qmax = (nrch + _NT - 1) // _NT
    nsub = _CE // _SUB
    has_x = xtable is not None
    nout = num_hslabs + (2 if has_x else 0)
    te_x = e_pad // (2 * _NT)
    nch_x = te_x // _CE

    mesh = plsc.VectorSubcoreMesh(core_axis_name="c", subcore_axis_name="s")

    def body(*refs):
        if has_x:
            (h_hbm, x_hbm, src_hbm, dst_hbm, ew_hbm, z_hbm, out_hbm,
             src2, idx2, dst3, ew2, rows2, zbuf, acc,
             sem_st, sem_g, sem_sc) = refs
        else:
            (h_hbm, src_hbm, dst_hbm, ew_hbm, z_hbm, out_hbm,
             src2, idx2, dst3, ew2, rows2, zbuf, acc,
             sem_st, sem_g, sem_sc) = refs
            x_hbm = None
        core = lax.axis_index("c")
        tile = lax.axis_index("s")
        pltpu.sync_copy(z_hbm, zbuf)

        def stage_issue(c, base0):
            base = base0 + c * _CE
            b = c % 2
            pltpu.async_copy(src_hbm.at[pl.ds(base, _CE)], src2.at[b], sem_st)
            pltpu.async_copy(ew_hbm.at[pl.ds(base, _CE)], ew2.at[b], sem_st)

        def stage_drain(c, base0):
            base = base0 + c * _CE
            b = c % 2
            pltpu.make_async_copy(
                src_hbm.at[pl.ds(base, _CE)], src2.at[b], sem_st).wait()
            pltpu.make_async_copy(
                ew_hbm.at[pl.ds(base, _CE)], ew2.at[b], sem_st).wait()

        def idx_compute(c, off):
            b = c % 2

            def f(i, car):
                idx2[b, pl.ds(i * _LANES, _LANES)] = (
                    src2[b, pl.ds(i * _LANES, _LANES)] * 4 + off)
                return car
            lax.fori_loop(0, _CE // _LANES, f, 0)

        def dst_stage(c, rbase0):
            pltpu.sync_copy(dst_hbm.at[pl.ds(rbase0 + c * nsub, nsub)],
                            dst3.at[c % 2])

        def gathers_issue(c, tbl, use_idx):
            b = c % 2
            iref = idx2 if use_idx else src2
            for j in range(nsub):
                pltpu.async_copy(
                    tbl.at[iref.at[b].at[pl.ds(j * _SUB, _SUB)]],
                    rows2.at[b].at[pl.ds(j * _SUB, _SUB)], sem_g)

        def gathers_drain(c, tbl, use_idx):
            b = c % 2
            iref = idx2 if use_idx else src2
            for j in range(nsub):
                pltpu.make_async_copy(
                    tbl.at[iref.at[b].at[pl.ds(j * _SUB, _SUB)]],
                    rows2.at[b].at[pl.ds(j * _SUB, _SUB)], sem_g).wait()

        def scatters_issue(c):
            b = c % 2
            for j in range(nsub):
                pltpu.async_copy(rows2.at[b].at[pl.ds(j * _SUB, _SUB)],
                                 acc.at[dst3.at[b].at[j]], sem_sc, add=True)

        def scatters_drain(c):
            b = c % 2
            for j in range(nsub):
                pltpu.make_async_copy(rows2.at[b].at[pl.ds(j * _SUB, _SUB)],
                                      acc.at[dst3.at[b].at[j]], sem_sc).wait()

        def multiply(c):
            b = c % 2

            def mg(g, car):
                ew16 = ew2[b, pl.ds(g * _LANES, _LANES)]
                for e in range(_LANES):
                    ea = g * _LANES + e
                    w = _splat(ew16, e)
                    rows2[b, ea, pl.ds(0, _LANES)] = (
                        rows2[b, ea, pl.ds(0, _LANES)] * w)
                return car
            lax.fori_loop(0, _CE // _LANES, mg, 0)

        def sweep(tbl, base0, rbase0, nch_l, off, use_idx):
            stage_issue(0, base0)
            stage_drain(0, base0)
            if use_idx:
                idx_compute(0, off)
            dst_stage(0, rbase0)
            gathers_issue(0, tbl, use_idx)
            stage_issue(1, base0)

            def step(i, car):
                nxt = i + 1

                @pl.when(nxt < nch_l)
                def _pre():
                    stage_drain(nxt, base0)
                    if use_idx:
                        idx_compute(nxt, off)

                    @pl.when(i >= 1)
                    def _ds():
                        scatters_drain(nxt)
                    dst_stage(nxt, rbase0)
                    gathers_issue(nxt, tbl, use_idx)

                gathers_drain(i, tbl, use_idx)
                multiply(i)
                scatters_issue(i)

                @pl.when(i + 2 < nch_l)
                def _st():
                    stage_issue(i + 2, base0)
                return car
            lax.fori_loop(0, nch_l, step, 0)
            scatters_drain(nch_l - 2)
            scatters_drain(nch_l - 1)

        def zero_acc():
            def zc(q, car):
                c = tile + q * _NT

                @pl.when(c < nrch)
                def _():
                    pltpu.sync_copy(zbuf, acc.at[pl.ds(c * _RCH, _RCH)])
                return car
            lax.fori_loop(0, qmax, zc, 0)

        def flush(sidx):
            def fc(q, car):
                c = tile + q * _NT

                @pl.when(c < nrch)
                def _():
                    pltpu.sync_copy(acc.at[pl.ds(c * _RCH, _RCH)],
                                    out_hbm.at[sidx].at[pl.ds(c * _RCH, _RCH)])
                return car
            lax.fori_loop(0, qmax, fc, 0)

        for k in range(num_hslabs // 2):
            s = 2 * k + core
            off = jnp.where(s >= 4, s - 4 + 4 * n_nodes, s).astype(jnp.int32)
            zero_acc()
            plsc.subcore_barrier()
            sweep(h_hbm, tile * te, tile * (te // _SUB), nch, off, True)
            plsc.subcore_barrier()
            flush(s)
            plsc.subcore_barrier()

        if has_x:
            base0 = core * (e_pad // 2) + tile * te_x
            rbase0 = core * (e_pad // (2 * _SUB)) + tile * (te_x // _SUB)
            zero_acc()
            plsc.subcore_barrier()
            sweep(x_hbm, base0, rbase0, nch_x, None, False)
            plsc.subcore_barrier()
            flush(num_hslabs + core)
            plsc.subcore_barrier()

    args = (htable, xtable, src_flat, dst2, ew_flat, zrows) if has_x else (
        htable, src_flat, dst2, ew_flat, zrows)
    return pl.kernel(
        body,
        out_type=jax.ShapeDtypeStruct((nout, n_pad, _WS), F32),
        mesh=mesh,
        compiler_params=pltpu.CompilerParams(use_tc_tiling_on_sc=False),
        scratch_types=[
            pltpu.VMEM((2, _CE), jnp.int32),
            pltpu.VMEM((2, _CE), jnp.int32),
            pltpu.VMEM((2, _CE // _SUB, _SUB), jnp.int32),
            pltpu.VMEM((2, _CE), F32),
            pltpu.VMEM((2, _CE, _WS), F32),
            pltpu.VMEM((_RCH, _WS), F32),
            pltpu.VMEM_SHARED((n_pad, _WS), F32),
            pltpu.SemaphoreType.DMA,
            pltpu.SemaphoreType.DMA,
            pltpu.SemaphoreType.DMA,
        ],
    )(*args)


def _ln_b(v, g, b):
    mu = jnp.mean(v, axis=-1, keepdims=True)
    var = jnp.mean((v - mu) ** 2, axis=-1, keepdims=True)
    return (v - mu) * lax.rsqrt(var + 1e-5) * g + b


def _tc_layer0(s0, c0, ws, bs, gh, bh, gc, bcc, tn):
    n = s0.shape[0]
    kdim = s0.shape[1]
    grid = n // tn

    def body(s_ref, c_ref, wi, wf, wc, wo, bi, bf, bc_, bo,
             gh_r, bh_r, gc_r, bcc_r, hid_o, cel_o):
        s = s_ref[...]
        pi = jnp.dot(s, wi[...], preferred_element_type=F32) + bi[...]
        pf = jnp.dot(s, wf[...], preferred_element_type=F32) + bf[...]
        pc = jnp.dot(s, wc[...], preferred_element_type=F32) + bc_[...]
        po = jnp.dot(s, wo[...], preferred_element_type=F32) + bo[...]
        i_ = jax.nn.sigmoid(pi)
        f_ = jax.nn.sigmoid(pf)
        g_ = jnp.tanh(pc)
        o_ = jax.nn.sigmoid(po)
        cn = f_ * c_ref[...] + i_ * g_
        hn = o_ * jnp.tanh(cn)
        hid_o[...] = _ln_b(hn, gh_r[...], bh_r[...])
        cel_o[...] = _ln_b(cn, gc_r[...], bcc_r[...])

    row = lambda i: (i, 0)
    fix = lambda i: (0, 0)
    return pl.pallas_call(
        body,
        grid=(grid,),
        in_specs=[
            pl.BlockSpec((tn, kdim), row), pl.BlockSpec((tn, 64), row),
        ] + [pl.BlockSpec((kdim, 64), fix)] * 4
          + [pl.BlockSpec((1, 64), fix)] * 8,
        out_specs=[pl.BlockSpec((tn, 64), row)] * 2,
        out_shape=[jax.ShapeDtypeStruct((n, 64), F32)] * 2,
    )(s0, c0, *ws, *bs, gh, bh, gc, bcc)


def _tc_layer1(s1, c1, skip, ws, bs, gh, bh, gc, bcc, go, bo_ln,
               fa, fb, fbias, f2w, f2b, tn):
    n = s1.shape[0]
    kdim = s1.shape[1]
    grid = n // tn

    def body(s_ref, c_ref, sk_ref, wi, wf, wc, wo, bi, bf, bc_, bo,
             gh_r, bh_r, gc_r, bcc_r, go_r, bol_r,
             fa_r, fb_r, fbias_r, f2w_r, f2b_r,
             hid_o, cel_o, o_out):
        s = s_ref[...]
        pi = jnp.dot(s, wi[...], preferred_element_type=F32) + bi[...]
        pf = jnp.dot(s, wf[...], preferred_element_type=F32) + bf[...]
        pc = jnp.dot(s, wc[...], preferred_element_type=F32) + bc_[...]
        po = jnp.dot(s, wo[...], preferred_element_type=F32) + bo[...]
        i_ = jax.nn.sigmoid(pi)
        f_ = jax.nn.sigmoid(pf)
        g_ = jnp.tanh(pc)
        o_ = jax.nn.sigmoid(po)
        cn = f_ * c_ref[...] + i_ * g_
        hn = o_ * jnp.tanh(cn)
        hid_o[...] = _ln_b(hn, gh_r[...], bh_r[...])
        cel_o[...] = _ln_b(cn, gc_r[...], bcc_r[...])
        ob = jnp.maximum(_ln_b(hn, go_r[...], bol_r[...]), 0.0)
        t = (jnp.dot(ob, fa_r[...], preferred_element_type=F32)
             + sk_ref[...] * fb_r[...] + fbias_r[...])
        t = jnp.maximum(t, 0.0)
        o_out[...] = jax.nn.sigmoid(
            jnp.sum(t * f2w_r[...], axis=-1, keepdims=True) + f2b_r[...])

    row = lambda i: (i, 0)
    fix = lambda i: (0, 0)
    return pl.pallas_call(
        body,
        grid=(grid,),
        in_specs=[
            pl.BlockSpec((tn, kdim), row), pl.BlockSpec((tn, 64), row),
            pl.BlockSpec((tn, 1), row),
        ] + [pl.BlockSpec((kdim, 64), fix)] * 4
          + [pl.BlockSpec((1, 64), fix)] * 10
          + [pl.BlockSpec((64, 64), fix)]
          + [pl.BlockSpec((1, 64), fix)] * 3
          + [pl.BlockSpec((1, 1), fix)],
        out_specs=[pl.BlockSpec((tn, 64), row)] * 2 + [pl.BlockSpec((tn, 1), row)],
        out_shape=[jax.ShapeDtypeStruct((n, 64), F32)] * 2
                  + [jax.ShapeDtypeStruct((n, 1), F32)],
    )(s1, c1, skip, *ws, *bs, gh, bh, gc, bcc, go, bo_ln,
      fa, fb, fbias, f2w, f2b)


def kernel(x, edge_index, edge_weight, skip, H, C, params):
    n = x.shape[0]
    e = edge_index.shape[1]
    fin = x.shape[1]
    n_pad = ((n + _RCH - 1) // _RCH) * _RCH
    e_blk = 2 * _NT * _CE
    e_pad = ((e + e_blk - 1) // e_blk) * e_blk
    padw = e_pad - e

    src = edge_index[0]
    dst = edge_index[1]
    srcf = jnp.concatenate([src, jnp.zeros((padw,), jnp.int32)])
    dstf = jnp.concatenate([dst, jnp.zeros((padw,), jnp.int32)])
    ewf = jnp.concatenate([edge_weight, jnp.zeros((padw,), F32)])
    dst2 = dstf.reshape(-1, _SUB)
    zrows = jnp.zeros((_RCH, _WS), F32)

    Hr = H.reshape(-1, _WS)
    xp = jnp.pad(x, ((0, 0), (0, _WS - fin)))
    S = _sc_spmm(Hr, xp, srcf, dst2, ewf, zrows, 8, n, n_pad)
    ah0 = jnp.concatenate([S[j, :n] for j in range(4)], axis=1)
    ah1 = jnp.concatenate([S[j, :n] for j in range(4, 8)], axis=1)
    ax = S[8, :n, :fin] + S[9, :n, :fin]
    s0cat = jnp.concatenate([ax, ah0], axis=1)

    l0, l1 = params['layers'][0], params['layers'][1]
    gates = ('i', 'f', 'c', 'o')
    ws0 = [jnp.concatenate([l0['Wx_' + g], l0['Wh_' + g]], axis=0) for g in gates]
    bs0 = [l0['b_' + g].reshape(1, 64) for g in gates]
    gh = params['ln_h_g'].reshape(1, 64)
    bh = params['ln_h_b'].reshape(1, 64)
    gc = params['ln_c_g'].reshape(1, 64)
    bcc = params['ln_c_b'].reshape(1, 64)

    tn = 2000
    hid0, cel0 = _tc_layer0(s0cat, C[0], ws0, bs0, gh, bh, gc, bcc, tn)

    hid0r = hid0.reshape(-1, _WS)
    S2 = _sc_spmm(hid0r, None, srcf, dst2, ewf, zrows, 4, n, n_pad)
    s1cat = jnp.concatenate([S2[j, :n] for j in range(4)] + [ah1], axis=1)

    ws1 = [jnp.concatenate([l1['Wx_' + g], l1['Wh_' + g]], axis=0) for g in gates]
    bs1 = [l1['b_' + g].reshape(1, 64) for g in gates]
    go = params['ln_o_g'].reshape(1, 64)
    bo_ln = params['ln_o_b'].reshape(1, 64)
    fa = params['fc1_W'][:64]
    fb = params['fc1_W'][64:65]
    fbias = params['fc1_b'].reshape(1, 64)
    f2w = params['fc2_W'].T
    f2b = params['fc2_b'].reshape(1, 1)

    hid1, cel1, o = _tc_layer1(s1cat, C[1], skip, ws1, bs1, gh, bh, gc, bcc,
                               go, bo_ln, fa, fb, fbias, f2w, f2b, tn)

    hidden = jnp.stack([hid0, hid1])
    cell = jnp.stack([cel0, cel1])
    return o, hidden, cell


# speedup vs baseline: 1.0312x; 1.0312x over previous
import jax
import jax.numpy as jnp
from jax import lax
from jax.experimental import pallas as pl
from jax.experimental.pallas import tpu as pltpu
from jax.experimental.pallas import tpu_sc as plsc

F32 = jnp.float32

_NT = 16
_LANES = 16
_WS = 16
_CE = 1024
_SUB = 128
_RCH = 512


def _splat(v16, e):
    return lax.gather(
        v16, jnp.full((_LANES, 1), e, jnp.int32),
        dimension_numbers=lax.GatherDimensionNumbers(
            offset_dims=(), collapsed_slice_dims=(0,), start_index_map=(0,)),
        slice_sizes=(1,),
        mode=lax.GatherScatterMode.PROMISE_IN_BOUNDS)


def _sc_spmm(htable, xtable, src_flat, dst2, ew_flat, zrows,
             num_hslabs, n_nodes, n_pad):
    e_pad = src_flat.shape[0]
    te = e_pad // _NT
    nch = te // _CE
    nrch = n_pad // _RCH
    qmax = (nrch + _NT - 1) // _NT
    nsub = _CE // _SUB
    has_x = xtable is not None
    nout = num_hslabs + (2 if has_x else 0)
    te_x = e_pad // (2 * _NT)
    nch_x = te_x // _CE

    mesh = plsc.VectorSubcoreMesh(core_axis_name="c", subcore_axis_name="s")

    def body(*refs):
        if has_x:
            (h_hbm, x_hbm, src_hbm, dst_hbm, ew_hbm, z_hbm, out_hbm,
             src2, idx2, dst3, ew2, rows2, zbuf, acc,
             sem_st, sem_g, sem_sc, sem_d) = refs
        else:
            (h_hbm, src_hbm, dst_hbm, ew_hbm, z_hbm, out_hbm,
             src2, idx2, dst3, ew2, rows2, zbuf, acc,
             sem_st, sem_g, sem_sc, sem_d) = refs
            x_hbm = None
        core = lax.axis_index("c")
        tile = lax.axis_index("s")
        pltpu.sync_copy(z_hbm, zbuf)

        def stage_issue(c, base0):
            base = base0 + c * _CE
            b = c % 2
            pltpu.async_copy(src_hbm.at[pl.ds(base, _CE)], src2.at[b], sem_st)
            pltpu.async_copy(ew_hbm.at[pl.ds(base, _CE)], ew2.at[b], sem_st)

        def stage_drain(c, base0):
            base = base0 + c * _CE
            b = c % 2
            pltpu.make_async_copy(
                src_hbm.at[pl.ds(base, _CE)], src2.at[b], sem_st).wait()
            pltpu.make_async_copy(
                ew_hbm.at[pl.ds(base, _CE)], ew2.at[b], sem_st).wait()

        def dst_issue(c, rbase0):
            pltpu.async_copy(dst_hbm.at[pl.ds(rbase0 + c * nsub, nsub)],
                             dst3.at[c % 2], sem_d)

        def dst_wait(c, rbase0):
            pltpu.make_async_copy(
                dst_hbm.at[pl.ds(rbase0 + c * nsub, nsub)],
                dst3.at[c % 2], sem_d).wait()

        def idx_compute(c, off):
            b = c % 2

            def f(i, car):
                idx2[b, pl.ds(i * _LANES, _LANES)] = (
                    src2[b, pl.ds(i * _LANES, _LANES)] * 4 + off)
                return car
            lax.fori_loop(0, _CE // _LANES, f, 0)

        def dst_stage(c, rbase0):
            pltpu.sync_copy(dst_hbm.at[pl.ds(rbase0 + c * nsub, nsub)],
                            dst3.at[c % 2])

        def gathers_issue(c, tbl, use_idx):
            b = c % 2
            iref = idx2 if use_idx else src2
            for j in range(nsub):
                pltpu.async_copy(
                    tbl.at[iref.at[b].at[pl.ds(j * _SUB, _SUB)]],
                    rows2.at[b].at[pl.ds(j * _SUB, _SUB)], sem_g)

        def gathers_drain(c, tbl, use_idx):
            b = c % 2
            iref = idx2 if use_idx else src2
            for j in range(nsub):
                pltpu.make_async_copy(
                    tbl.at[iref.at[b].at[pl.ds(j * _SUB, _SUB)]],
                    rows2.at[b].at[pl.ds(j * _SUB, _SUB)], sem_g).wait()

        def scatters_issue(c):
            b = c % 2
            for j in range(nsub):
                pltpu.async_copy(rows2.at[b].at[pl.ds(j * _SUB, _SUB)],
                                 acc.at[dst3.at[b].at[j]], sem_sc, add=True)

        def scatters_drain(c):
            b = c % 2
            for j in range(nsub):
                pltpu.make_async_copy(rows2.at[b].at[pl.ds(j * _SUB, _SUB)],
                                      acc.at[dst3.at[b].at[j]], sem_sc).wait()

        def multiply(c):
            b = c % 2

            def mg(g, car):
                ew16 = ew2[b, pl.ds(g * _LANES, _LANES)]
                for e in range(_LANES):
                    ea = g * _LANES + e
                    w = _splat(ew16, e)
                    rows2[b, ea, pl.ds(0, _LANES)] = (
                        rows2[b, ea, pl.ds(0, _LANES)] * w)
                return car
            lax.fori_loop(0, _CE // _LANES, mg, 0)

        def sweep(tbl, base0, rbase0, nch_l, off, use_idx):
            stage_issue(0, base0)
            dst_issue(0, rbase0)
            dst_issue(1, rbase0)
            stage_drain(0, base0)
            if use_idx:
                idx_compute(0, off)
            gathers_issue(0, tbl, use_idx)
            stage_issue(1, base0)

            def step(i, car):
                nxt = i + 1

                @pl.when(nxt < nch_l)
                def _pre():
                    stage_drain(nxt, base0)
                    if use_idx:
                        idx_compute(nxt, off)

                    @pl.when(i >= 1)
                    def _ds():
                        scatters_drain(nxt)
                        dst_issue(nxt, rbase0)
                    gathers_issue(nxt, tbl, use_idx)

                gathers_drain(i, tbl, use_idx)
                multiply(i)
                dst_wait(i, rbase0)
                scatters_issue(i)

                @pl.when(i + 2 < nch_l)
                def _st():
                    stage_issue(i + 2, base0)
                return car
            lax.fori_loop(0, nch_l, step, 0)
            scatters_drain(nch_l - 2)
            scatters_drain(nch_l - 1)

        def zero_acc():
            def zc(q, car):
                c = tile + q * _NT

                @pl.when(c < nrch)
                def _():
                    pltpu.sync_copy(zbuf, acc.at[pl.ds(c * _RCH, _RCH)])
                return car
            lax.fori_loop(0, qmax, zc, 0)

        def flush(sidx):
            def fc(q, car):
                c = tile + q * _NT

                @pl.when(c < nrch)
                def _():
                    pltpu.sync_copy(acc.at[pl.ds(c * _RCH, _RCH)],
                                    out_hbm.at[sidx].at[pl.ds(c * _RCH, _RCH)])
                return car
            lax.fori_loop(0, qmax, fc, 0)

        for k in range(num_hslabs // 2):
            s = 2 * k + core
            off = jnp.where(s >= 4, s - 4 + 4 * n_nodes, s).astype(jnp.int32)
            zero_acc()
            plsc.subcore_barrier()
            sweep(h_hbm, tile * te, tile * (te // _SUB), nch, off, True)
            plsc.subcore_barrier()
            flush(s)
            plsc.subcore_barrier()

        if has_x:
            base0 = core * (e_pad // 2) + tile * te_x
            rbase0 = core * (e_pad // (2 * _SUB)) + tile * (te_x // _SUB)
            zero_acc()
            plsc.subcore_barrier()
            sweep(x_hbm, base0, rbase0, nch_x, None, False)
            plsc.subcore_barrier()
            flush(num_hslabs + core)
            plsc.subcore_barrier()

    args = (htable, xtable, src_flat, dst2, ew_flat, zrows) if has_x else (
        htable, src_flat, dst2, ew_flat, zrows)
    return pl.kernel(
        body,
        out_type=jax.ShapeDtypeStruct((nout, n_pad, _WS), F32),
        mesh=mesh,
        compiler_params=pltpu.CompilerParams(use_tc_tiling_on_sc=False),
        scratch_types=[
            pltpu.VMEM((2, _CE), jnp.int32),
            pltpu.VMEM((2, _CE), jnp.int32),
            pltpu.VMEM((2, _CE // _SUB, _SUB), jnp.int32),
            pltpu.VMEM((2, _CE), F32),
            pltpu.VMEM((2, _CE, _WS), F32),
            pltpu.VMEM((_RCH, _WS), F32),
            pltpu.VMEM_SHARED((n_pad, _WS), F32),
            pltpu.SemaphoreType.DMA,
            pltpu.SemaphoreType.DMA,
            pltpu.SemaphoreType.DMA,
            pltpu.SemaphoreType.DMA,
        ],
    )(*args)


def _ln_b(v, g, b):
    mu = jnp.mean(v, axis=-1, keepdims=True)
    var = jnp.mean((v - mu) ** 2, axis=-1, keepdims=True)
    return (v - mu) * lax.rsqrt(var + 1e-5) * g + b


def _tc_layer0(s0, c0, ws, bs, gh, bh, gc, bcc, tn):
    n = s0.shape[0]
    kdim = s0.shape[1]
    grid = n // tn

    def body(s_ref, c_ref, wi, wf, wc, wo, bi, bf, bc_, bo,
             gh_r, bh_r, gc_r, bcc_r, hid_o, cel_o):
        s = s_ref[...]
        pi = jnp.dot(s, wi[...], preferred_element_type=F32) + bi[...]
        pf = jnp.dot(s, wf[...], preferred_element_type=F32) + bf[...]
        pc = jnp.dot(s, wc[...], preferred_element_type=F32) + bc_[...]
        po = jnp.dot(s, wo[...], preferred_element_type=F32) + bo[...]
        i_ = jax.nn.sigmoid(pi)
        f_ = jax.nn.sigmoid(pf)
        g_ = jnp.tanh(pc)
        o_ = jax.nn.sigmoid(po)
        cn = f_ * c_ref[...] + i_ * g_
        hn = o_ * jnp.tanh(cn)
        hid_o[...] = _ln_b(hn, gh_r[...], bh_r[...])
        cel_o[...] = _ln_b(cn, gc_r[...], bcc_r[...])

    row = lambda i: (i, 0)
    fix = lambda i: (0, 0)
    return pl.pallas_call(
        body,
        grid=(grid,),
        in_specs=[
            pl.BlockSpec((tn, kdim), row), pl.BlockSpec((tn, 64), row),
        ] + [pl.BlockSpec((kdim, 64), fix)] * 4
          + [pl.BlockSpec((1, 64), fix)] * 8,
        out_specs=[pl.BlockSpec((tn, 64), row)] * 2,
        out_shape=[jax.ShapeDtypeStruct((n, 64), F32)] * 2,
    )(s0, c0, *ws, *bs, gh, bh, gc, bcc)


def _tc_layer1(s1, c1, skip, ws, bs, gh, bh, gc, bcc, go, bo_ln,
               fa, fb, fbias, f2w, f2b, tn):
    n = s1.shape[0]
    kdim = s1.shape[1]
    grid = n // tn

    def body(s_ref, c_ref, sk_ref, wi, wf, wc, wo, bi, bf, bc_, bo,
             gh_r, bh_r, gc_r, bcc_r, go_r, bol_r,
             fa_r, fb_r, fbias_r, f2w_r, f2b_r,
             hid_o, cel_o, o_out):
        s = s_ref[...]
        pi = jnp.dot(s, wi[...], preferred_element_type=F32) + bi[...]
        pf = jnp.dot(s, wf[...], preferred_element_type=F32) + bf[...]
        pc = jnp.dot(s, wc[...], preferred_element_type=F32) + bc_[...]
        po = jnp.dot(s, wo[...], preferred_element_type=F32) + bo[...]
        i_ = jax.nn.sigmoid(pi)
        f_ = jax.nn.sigmoid(pf)
        g_ = jnp.tanh(pc)
        o_ = jax.nn.sigmoid(po)
        cn = f_ * c_ref[...] + i_ * g_
        hn = o_ * jnp.tanh(cn)
        hid_o[...] = _ln_b(hn, gh_r[...], bh_r[...])
        cel_o[...] = _ln_b(cn, gc_r[...], bcc_r[...])
        ob = jnp.maximum(_ln_b(hn, go_r[...], bol_r[...]), 0.0)
        t = (jnp.dot(ob, fa_r[...], preferred_element_type=F32)
             + sk_ref[...] * fb_r[...] + fbias_r[...])
        t = jnp.maximum(t, 0.0)
        o_out[...] = jax.nn.sigmoid(
            jnp.sum(t * f2w_r[...], axis=-1, keepdims=True) + f2b_r[...])

    row = lambda i: (i, 0)
    fix = lambda i: (0, 0)
    return pl.pallas_call(
        body,
        grid=(grid,),
        in_specs=[
            pl.BlockSpec((tn, kdim), row), pl.BlockSpec((tn, 64), row),
            pl.BlockSpec((tn, 1), row),
        ] + [pl.BlockSpec((kdim, 64), fix)] * 4
          + [pl.BlockSpec((1, 64), fix)] * 10
          + [pl.BlockSpec((64, 64), fix)]
          + [pl.BlockSpec((1, 64), fix)] * 3
          + [pl.BlockSpec((1, 1), fix)],
        out_specs=[pl.BlockSpec((tn, 64), row)] * 2 + [pl.BlockSpec((tn, 1), row)],
        out_shape=[jax.ShapeDtypeStruct((n, 64), F32)] * 2
                  + [jax.ShapeDtypeStruct((n, 1), F32)],
    )(s1, c1, skip, *ws, *bs, gh, bh, gc, bcc, go, bo_ln,
      fa, fb, fbias, f2w, f2b)


def kernel(x, edge_index, edge_weight, skip, H, C, params):
    n = x.shape[0]
    e = edge_index.shape[1]
    fin = x.shape[1]
    n_pad = ((n + _RCH - 1) // _RCH) * _RCH
    e_blk = 2 * _NT * _CE
    e_pad = ((e + e_blk - 1) // e_blk) * e_blk
    padw = e_pad - e

    src = edge_index[0]
    dst = edge_index[1]
    srcf = jnp.concatenate([src, jnp.zeros((padw,), jnp.int32)])
    dstf = jnp.concatenate([dst, jnp.zeros((padw,), jnp.int32)])
    ewf = jnp.concatenate([edge_weight, jnp.zeros((padw,), F32)])
    dst2 = dstf.reshape(-1, _SUB)
    zrows = jnp.zeros((_RCH, _WS), F32)

    Hr = H.reshape(-1, _WS)
    xp = jnp.pad(x, ((0, 0), (0, _WS - fin)))
    S = _sc_spmm(Hr, xp, srcf, dst2, ewf, zrows, 8, n, n_pad)
    ah0 = jnp.concatenate([S[j, :n] for j in range(4)], axis=1)
    ah1 = jnp.concatenate([S[j, :n] for j in range(4, 8)], axis=1)
    ax = S[8, :n, :fin] + S[9, :n, :fin]
    s0cat = jnp.concatenate([ax, ah0], axis=1)

    l0, l1 = params['layers'][0], params['layers'][1]
    gates = ('i', 'f', 'c', 'o')
    ws0 = [jnp.concatenate([l0['Wx_' + g], l0['Wh_' + g]], axis=0) for g in gates]
    bs0 = [l0['b_' + g].reshape(1, 64) for g in gates]
    gh = params['ln_h_g'].reshape(1, 64)
    bh = params['ln_h_b'].reshape(1, 64)
    gc = params['ln_c_g'].reshape(1, 64)
    bcc = params['ln_c_b'].reshape(1, 64)

    tn = 2000
    hid0, cel0 = _tc_layer0(s0cat, C[0], ws0, bs0, gh, bh, gc, bcc, tn)

    hid0r = hid0.reshape(-1, _WS)
    S2 = _sc_spmm(hid0r, None, srcf, dst2, ewf, zrows, 4, n, n_pad)
    s1cat = jnp.concatenate([S2[j, :n] for j in range(4)] + [ah1], axis=1)

    ws1 = [jnp.concatenate([l1['Wx_' + g], l1['Wh_' + g]], axis=0) for g in gates]
    bs1 = [l1['b_' + g].reshape(1, 64) for g in gates]
    go = params['ln_o_g'].reshape(1, 64)
    bo_ln = params['ln_o_b'].reshape(1, 64)
    fa = params['fc1_W'][:64]
    fb = params['fc1_W'][64:65]
    fbias = params['fc1_b'].reshape(1, 64)
    f2w = params['fc2_W'].T
    f2b = params['fc2_b'].reshape(1, 1)

    hid1, cel1, o = _tc_layer1(s1cat, C[1], skip, ws1, bs1, gh, bh, gc, bcc,
                               go, bo_ln, fa, fb, fbias, f2w, f2b, tn)

    hidden = jnp.stack([hid0, hid1])
    cell = jnp.stack([cel0, cel1])
    return o, hidden, cell


# speedup vs baseline: 1.0909x; 1.0579x over previous
import jax
import jax.numpy as jnp
from jax import lax
from jax.experimental import pallas as pl
from jax.experimental.pallas import tpu as pltpu
from jax.experimental.pallas import tpu_sc as plsc

F32 = jnp.float32

_NT = 16
_LANES = 16
_WS = 16
_CE = 1024
_SUB = 128
_RCH = 512


def _splat(v16, e):
    return lax.gather(
        v16, jnp.full((_LANES, 1), e, jnp.int32),
        dimension_numbers=lax.GatherDimensionNumbers(
            offset_dims=(), collapsed_slice_dims=(0,), start_index_map=(0,)),
        slice_sizes=(1,),
        mode=lax.GatherScatterMode.PROMISE_IN_BOUNDS)


def _sc_spmm(htable, xtable, src_flat, dst2, ew_flat, zrows,
             num_hslabs, n_nodes, n_pad):
    e_pad = src_flat.shape[0]
    te = e_pad // _NT
    nch = te // _CE
    nrch = n_pad // _RCH
    qmax = (nrch + _NT - 1) // _NT
    nsub = _CE // _SUB
    has_x = xtable is not None
    nout = num_hslabs + (2 if has_x else 0)
    te_x = e_pad // (2 * _NT)
    nch_x = te_x // _CE

    mesh = plsc.VectorSubcoreMesh(core_axis_name="c", subcore_axis_name="s")

    def body(*refs):
        if has_x:
            (h_hbm, x_hbm, src_hbm, dst_hbm, ew_hbm, z_hbm, out_hbm,
             src2, idx2, dst3, ew2, rows2, zbuf, acc,
             sem_st, sem_g, sem_sc, sem_d) = refs
        else:
            (h_hbm, src_hbm, dst_hbm, ew_hbm, z_hbm, out_hbm,
             src2, idx2, dst3, ew2, rows2, zbuf, acc,
             sem_st, sem_g, sem_sc, sem_d) = refs
            x_hbm = None
        core = lax.axis_index("c")
        tile = lax.axis_index("s")
        pltpu.sync_copy(z_hbm, zbuf)

        def stage_issue(c, base0):
            base = base0 + c * _CE
            b = c % 2
            pltpu.async_copy(src_hbm.at[pl.ds(base, _CE)], src2.at[b], sem_st)
            pltpu.async_copy(ew_hbm.at[pl.ds(base, _CE)], ew2.at[b], sem_st)

        def stage_drain(c, base0):
            base = base0 + c * _CE
            b = c % 2
            pltpu.make_async_copy(
                src_hbm.at[pl.ds(base, _CE)], src2.at[b], sem_st).wait()
            pltpu.make_async_copy(
                ew_hbm.at[pl.ds(base, _CE)], ew2.at[b], sem_st).wait()

        def dst_issue(c, rbase0):
            pltpu.async_copy(dst_hbm.at[pl.ds(rbase0 + c * nsub, nsub)],
                             dst3.at[c % 2], sem_d)

        def dst_wait(c, rbase0):
            pltpu.make_async_copy(
                dst_hbm.at[pl.ds(rbase0 + c * nsub, nsub)],
                dst3.at[c % 2], sem_d).wait()

        def idx_compute(c, off):
            b = c % 2

            def f(i, car):
                idx2[b, pl.ds(i * _LANES, _LANES)] = (
                    src2[b, pl.ds(i * _LANES, _LANES)] * 4 + off)
                return car
            lax.fori_loop(0, _CE // _LANES, f, 0)

        def dst_stage(c, rbase0):
            pltpu.sync_copy(dst_hbm.at[pl.ds(rbase0 + c * nsub, nsub)],
                            dst3.at[c % 2])

        def gathers_issue(c, tbl, use_idx):
            b = c % 2
            iref = idx2 if use_idx else src2
            for j in range(nsub):
                pltpu.async_copy(
                    tbl.at[iref.at[b].at[pl.ds(j * _SUB, _SUB)]],
                    rows2.at[b].at[pl.ds(j * _SUB, _SUB)], sem_g)

        def gathers_drain(c, tbl, use_idx):
            b = c % 2
            iref = idx2 if use_idx else src2
            for j in range(nsub):
                pltpu.make_async_copy(
                    tbl.at[iref.at[b].at[pl.ds(j * _SUB, _SUB)]],
                    rows2.at[b].at[pl.ds(j * _SUB, _SUB)], sem_g).wait()

        def scatters_issue(c):
            b = c % 2
            for j in range(nsub):
                pltpu.async_copy(rows2.at[b].at[pl.ds(j * _SUB, _SUB)],
                                 acc.at[dst3.at[b].at[j]], sem_sc, add=True)

        def scatters_drain(c):
            b = c % 2
            for j in range(nsub):
                pltpu.make_async_copy(rows2.at[b].at[pl.ds(j * _SUB, _SUB)],
                                      acc.at[dst3.at[b].at[j]], sem_sc).wait()

        def multiply(c):
            b = c % 2

            def mg(g, car):
                ew16 = ew2[b, pl.ds(g * _LANES, _LANES)]
                for e in range(_LANES):
                    ea = g * _LANES + e
                    w = _splat(ew16, e)
                    rows2[b, ea, pl.ds(0, _LANES)] = (
                        rows2[b, ea, pl.ds(0, _LANES)] * w)
                return car
            lax.fori_loop(0, _CE // _LANES, mg, 0)

        def sweep(tbl, base0, rbase0, nch_l, off, use_idx):
            stage_issue(0, base0)
            dst_issue(0, rbase0)
            dst_issue(1, rbase0)
            stage_drain(0, base0)
            if use_idx:
                idx_compute(0, off)
            gathers_issue(0, tbl, use_idx)
            stage_issue(1, base0)

            def step(i, car):
                nxt = i + 1

                @pl.when(nxt < nch_l)
                def _pre():
                    stage_drain(nxt, base0)
                    if use_idx:
                        idx_compute(nxt, off)

                    @pl.when(i >= 1)
                    def _ds():
                        scatters_drain(nxt)
                        dst_issue(nxt, rbase0)
                    gathers_issue(nxt, tbl, use_idx)

                gathers_drain(i, tbl, use_idx)
                multiply(i)
                dst_wait(i, rbase0)
                scatters_issue(i)

                @pl.when(i + 2 < nch_l)
                def _st():
                    stage_issue(i + 2, base0)
                return car
            lax.fori_loop(0, nch_l, step, 0)
            scatters_drain(nch_l - 2)
            scatters_drain(nch_l - 1)

        def zero_acc():
            def zc(q, car):
                c = tile + q * _NT

                @pl.when(c < nrch)
                def _():
                    pltpu.sync_copy(zbuf, acc.at[pl.ds(c * _RCH, _RCH)])
                return car
            lax.fori_loop(0, qmax, zc, 0)

        def flush(sidx):
            def fc(q, car):
                c = tile + q * _NT

                @pl.when(c < nrch)
                def _():
                    pltpu.sync_copy(acc.at[pl.ds(c * _RCH, _RCH)],
                                    out_hbm.at[sidx].at[pl.ds(c * _RCH, _RCH)])
                return car
            lax.fori_loop(0, qmax, fc, 0)

        for k in range(num_hslabs // 2):
            s = 2 * k + core
            off = jnp.where(s >= 4, s - 4 + 4 * n_nodes, s).astype(jnp.int32)
            zero_acc()
            plsc.subcore_barrier()
            sweep(h_hbm, tile * te, tile * (te // _SUB), nch, off, True)
            plsc.subcore_barrier()
            flush(s)
            plsc.subcore_barrier()

        if has_x:
            base0 = core * (e_pad // 2) + tile * te_x
            rbase0 = core * (e_pad // (2 * _SUB)) + tile * (te_x // _SUB)
            zero_acc()
            plsc.subcore_barrier()
            sweep(x_hbm, base0, rbase0, nch_x, None, False)
            plsc.subcore_barrier()
            flush(num_hslabs + core)
            plsc.subcore_barrier()

    args = (htable, xtable, src_flat, dst2, ew_flat, zrows) if has_x else (
        htable, src_flat, dst2, ew_flat, zrows)
    return pl.kernel(
        body,
        out_type=jax.ShapeDtypeStruct((nout, n_pad, _WS), F32),
        mesh=mesh,
        compiler_params=pltpu.CompilerParams(use_tc_tiling_on_sc=False),
        scratch_types=[
            pltpu.VMEM((2, _CE), jnp.int32),
            pltpu.VMEM((2, _CE), jnp.int32),
            pltpu.VMEM((2, _CE // _SUB, _SUB), jnp.int32),
            pltpu.VMEM((2, _CE), F32),
            pltpu.VMEM((2, _CE, _WS), F32),
            pltpu.VMEM((_RCH, _WS), F32),
            pltpu.VMEM_SHARED((n_pad, _WS), F32),
            pltpu.SemaphoreType.DMA,
            pltpu.SemaphoreType.DMA,
            pltpu.SemaphoreType.DMA,
            pltpu.SemaphoreType.DMA,
        ],
    )(*args)


def _ln_b(v, g, b):
    mu = jnp.mean(v, axis=-1, keepdims=True)
    var = jnp.mean((v - mu) ** 2, axis=-1, keepdims=True)
    return (v - mu) * lax.rsqrt(var + 1e-5) * g + b


def _tc_layer0(s0, c0, ws, bs, gh, bh, gc, bcc, tn):
    n = s0.shape[0]
    kdim = s0.shape[1]
    grid = n // tn

    def body(s_ref, c_ref, wi, wf, wc, wo, bi, bf, bc_, bo,
             gh_r, bh_r, gc_r, bcc_r, hid_o, cel_o):
        s = s_ref[...]
        pi = jnp.dot(s, wi[...], preferred_element_type=F32) + bi[...]
        pf = jnp.dot(s, wf[...], preferred_element_type=F32) + bf[...]
        pc = jnp.dot(s, wc[...], preferred_element_type=F32) + bc_[...]
        po = jnp.dot(s, wo[...], preferred_element_type=F32) + bo[...]
        i_ = jax.nn.sigmoid(pi)
        f_ = jax.nn.sigmoid(pf)
        g_ = jnp.tanh(pc)
        o_ = jax.nn.sigmoid(po)
        cn = f_ * c_ref[...] + i_ * g_
        hn = o_ * jnp.tanh(cn)
        hid_o[...] = _ln_b(hn, gh_r[...], bh_r[...])
        cel_o[...] = _ln_b(cn, gc_r[...], bcc_r[...])

    row = lambda i: (i, 0)
    fix = lambda i: (0, 0)
    return pl.pallas_call(
        body,
        grid=(grid,),
        in_specs=[
            pl.BlockSpec((tn, kdim), row), pl.BlockSpec((tn, 64), row),
        ] + [pl.BlockSpec((kdim, 64), fix)] * 4
          + [pl.BlockSpec((1, 64), fix)] * 8,
        out_specs=[pl.BlockSpec((tn, 64), row)] * 2,
        out_shape=[jax.ShapeDtypeStruct((n, 64), F32)] * 2,
    )(s0, c0, *ws, *bs, gh, bh, gc, bcc)


def _tc_layer1(s1, c1, skip, ws, bs, gh, bh, gc, bcc, go, bo_ln,
               fa, fb, fbias, f2w, f2b, tn):
    n = s1.shape[0]
    kdim = s1.shape[1]
    grid = n // tn

    def body(s_ref, c_ref, sk_ref, wi, wf, wc, wo, bi, bf, bc_, bo,
             gh_r, bh_r, gc_r, bcc_r, go_r, bol_r,
             fa_r, fb_r, fbias_r, f2w_r, f2b_r,
             hid_o, cel_o, o_out):
        s = s_ref[...]
        pi = jnp.dot(s, wi[...], preferred_element_type=F32) + bi[...]
        pf = jnp.dot(s, wf[...], preferred_element_type=F32) + bf[...]
        pc = jnp.dot(s, wc[...], preferred_element_type=F32) + bc_[...]
        po = jnp.dot(s, wo[...], preferred_element_type=F32) + bo[...]
        i_ = jax.nn.sigmoid(pi)
        f_ = jax.nn.sigmoid(pf)
        g_ = jnp.tanh(pc)
        o_ = jax.nn.sigmoid(po)
        cn = f_ * c_ref[...] + i_ * g_
        hn = o_ * jnp.tanh(cn)
        hid_o[...] = _ln_b(hn, gh_r[...], bh_r[...])
        cel_o[...] = _ln_b(cn, gc_r[...], bcc_r[...])
        ob = jnp.maximum(_ln_b(hn, go_r[...], bol_r[...]), 0.0)
        t = (jnp.dot(ob, fa_r[...], preferred_element_type=F32)
             + sk_ref[...] * fb_r[...] + fbias_r[...])
        t = jnp.maximum(t, 0.0)
        o_out[...] = jax.nn.sigmoid(
            jnp.sum(t * f2w_r[...], axis=-1, keepdims=True) + f2b_r[...])

    row = lambda i: (i, 0)
    fix = lambda i: (0, 0)
    return pl.pallas_call(
        body,
        grid=(grid,),
        in_specs=[
            pl.BlockSpec((tn, kdim), row), pl.BlockSpec((tn, 64), row),
            pl.BlockSpec((tn, 1), row),
        ] + [pl.BlockSpec((kdim, 64), fix)] * 4
          + [pl.BlockSpec((1, 64), fix)] * 10
          + [pl.BlockSpec((64, 64), fix)]
          + [pl.BlockSpec((1, 64), fix)] * 3
          + [pl.BlockSpec((1, 1), fix)],
        out_specs=[pl.BlockSpec((tn, 64), row)] * 2 + [pl.BlockSpec((tn, 1), row)],
        out_shape=[jax.ShapeDtypeStruct((n, 64), F32)] * 2
                  + [jax.ShapeDtypeStruct((n, 1), F32)],
    )(s1, c1, skip, *ws, *bs, gh, bh, gc, bcc, go, bo_ln,
      fa, fb, fbias, f2w, f2b)


def kernel(x, edge_index, edge_weight, skip, H, C, params):
    n = x.shape[0]
    e = edge_index.shape[1]
    fin = x.shape[1]
    n_pad = ((n + _RCH - 1) // _RCH) * _RCH
    e_blk = 2 * _NT * _CE
    e_pad = ((e + e_blk - 1) // e_blk) * e_blk
    padw = e_pad - e

    src = edge_index[0]
    dst = edge_index[1]
    srcf = jnp.concatenate([src, jnp.zeros((padw,), jnp.int32)])
    dstf = jnp.concatenate([dst, jnp.zeros((padw,), jnp.int32)])
    ewf = jnp.concatenate([edge_weight, jnp.zeros((padw,), F32)])
    dst2 = dstf.reshape(-1, _SUB)
    zrows = jnp.zeros((_RCH, _WS), F32)

    H0r = H[0].reshape(-1, _WS)
    H1r = H[1].reshape(-1, _WS)
    xp = jnp.pad(x, ((0, 0), (0, _WS - fin)))
    S = _sc_spmm(H0r, xp, srcf, dst2, ewf, zrows, 4, n, n_pad)
    Sb = _sc_spmm(H1r, None, srcf, dst2, ewf, zrows, 4, n, n_pad)
    ah0 = jnp.concatenate([S[j, :n] for j in range(4)], axis=1)
    ah1 = jnp.concatenate([Sb[j, :n] for j in range(4)], axis=1)
    ax = S[4, :n, :fin] + S[5, :n, :fin]
    s0cat = jnp.concatenate([ax, ah0], axis=1)

    l0, l1 = params['layers'][0], params['layers'][1]
    gates = ('i', 'f', 'c', 'o')
    ws0 = [jnp.concatenate([l0['Wx_' + g], l0['Wh_' + g]], axis=0) for g in gates]
    bs0 = [l0['b_' + g].reshape(1, 64) for g in gates]
    gh = params['ln_h_g'].reshape(1, 64)
    bh = params['ln_h_b'].reshape(1, 64)
    gc = params['ln_c_g'].reshape(1, 64)
    bcc = params['ln_c_b'].reshape(1, 64)

    tn = 2000
    hid0, cel0 = _tc_layer0(s0cat, C[0], ws0, bs0, gh, bh, gc, bcc, tn)

    hid0r = hid0.reshape(-1, _WS)
    S2 = _sc_spmm(hid0r, None, srcf, dst2, ewf, zrows, 4, n, n_pad)
    s1cat = jnp.concatenate([S2[j, :n] for j in range(4)] + [ah1], axis=1)

    ws1 = [jnp.concatenate([l1['Wx_' + g], l1['Wh_' + g]], axis=0) for g in gates]
    bs1 = [l1['b_' + g].reshape(1, 64) for g in gates]
    go = params['ln_o_g'].reshape(1, 64)
    bo_ln = params['ln_o_b'].reshape(1, 64)
    fa = params['fc1_W'][:64]
    fb = params['fc1_W'][64:65]
    fbias = params['fc1_b'].reshape(1, 64)
    f2w = params['fc2_W'].T
    f2b = params['fc2_b'].reshape(1, 1)

    hid1, cel1, o = _tc_layer1(s1cat, C[1], skip, ws1, bs1, gh, bh, gc, bcc,
                               go, bo_ln, fa, fb, fbias, f2w, f2b, tn)

    hidden = jnp.stack([hid0, hid1])
    cell = jnp.stack([cel0, cel1])
    return o, hidden, cell


# speedup vs baseline: 1.2846x; 1.1775x over previous
import jax
import jax.numpy as jnp
from jax import lax
from jax.experimental import pallas as pl
from jax.experimental.pallas import tpu as pltpu
from jax.experimental.pallas import tpu_sc as plsc

F32 = jnp.float32

_NT = 16
_LANES = 16
_WS = 16
_CE = 1024
_SUB = 128
_RCH = 512


def _splat(v16, e):
    return lax.gather(
        v16, jnp.full((_LANES, 1), e, jnp.int32),
        dimension_numbers=lax.GatherDimensionNumbers(
            offset_dims=(), collapsed_slice_dims=(0,), start_index_map=(0,)),
        slice_sizes=(1,),
        mode=lax.GatherScatterMode.PROMISE_IN_BOUNDS)


def _sc_spmm(htable, xtable, src_flat, dst2, ew_flat, zrows,
             num_hslabs, n_nodes, n_pad):
    e_pad = src_flat.shape[0]
    te = e_pad // _NT
    nch = te // _CE
    nrch = n_pad // _RCH
    qmax = (nrch + _NT - 1) // _NT
    nsub = _CE // _SUB
    has_x = xtable is not None
    nout = num_hslabs + (2 if has_x else 0)
    te_x = e_pad // (2 * _NT)
    nch_x = te_x // _CE

    mesh = plsc.VectorSubcoreMesh(core_axis_name="c", subcore_axis_name="s")

    def body(*refs):
        if has_x:
            (h_hbm, x_hbm, src_hbm, dst_hbm, ew_hbm, z_hbm, out_hbm,
             src2, idx2, dst3, ew2, rows2, zbuf, acc,
             sem_st, sem_g, sem_sc, sem_d) = refs
        else:
            (h_hbm, src_hbm, dst_hbm, ew_hbm, z_hbm, out_hbm,
             src2, idx2, dst3, ew2, rows2, zbuf, acc,
             sem_st, sem_g, sem_sc, sem_d) = refs
            x_hbm = None
        core = lax.axis_index("c")
        tile = lax.axis_index("s")
        pltpu.sync_copy(z_hbm, zbuf)

        def stage_issue(c, base0):
            base = base0 + c * _CE
            b = c % 2
            pltpu.async_copy(src_hbm.at[pl.ds(base, _CE)], src2.at[b], sem_st)
            pltpu.async_copy(ew_hbm.at[pl.ds(base, _CE)], ew2.at[b], sem_st)

        def stage_drain(c, base0):
            base = base0 + c * _CE
            b = c % 2
            pltpu.make_async_copy(
                src_hbm.at[pl.ds(base, _CE)], src2.at[b], sem_st).wait()
            pltpu.make_async_copy(
                ew_hbm.at[pl.ds(base, _CE)], ew2.at[b], sem_st).wait()

        def dst_issue(c, rbase0):
            pltpu.async_copy(dst_hbm.at[pl.ds(rbase0 + c * nsub, nsub)],
                             dst3.at[c % 2], sem_d)

        def dst_wait(c, rbase0):
            pltpu.make_async_copy(
                dst_hbm.at[pl.ds(rbase0 + c * nsub, nsub)],
                dst3.at[c % 2], sem_d).wait()

        def idx_compute(c, off):
            b = c % 2

            def f(i, car):
                idx2[b, pl.ds(i * _LANES, _LANES)] = (
                    src2[b, pl.ds(i * _LANES, _LANES)] * 4 + off)
                return car
            lax.fori_loop(0, _CE // _LANES, f, 0)

        def dst_stage(c, rbase0):
            pltpu.sync_copy(dst_hbm.at[pl.ds(rbase0 + c * nsub, nsub)],
                            dst3.at[c % 2])

        def gathers_issue(c, tbl, use_idx):
            b = c % 2
            iref = idx2 if use_idx else src2
            for j in range(nsub):
                pltpu.async_copy(
                    tbl.at[iref.at[b].at[pl.ds(j * _SUB, _SUB)]],
                    rows2.at[b].at[pl.ds(j * _SUB, _SUB)], sem_g)

        def gathers_drain(c, tbl, use_idx):
            b = c % 2
            iref = idx2 if use_idx else src2
            for j in range(nsub):
                pltpu.make_async_copy(
                    tbl.at[iref.at[b].at[pl.ds(j * _SUB, _SUB)]],
                    rows2.at[b].at[pl.ds(j * _SUB, _SUB)], sem_g).wait()

        def scatters_issue(c):
            b = c % 2
            for j in range(nsub):
                pltpu.async_copy(rows2.at[b].at[pl.ds(j * _SUB, _SUB)],
                                 acc.at[dst3.at[b].at[j]], sem_sc, add=True)

        def scatters_drain(c):
            b = c % 2
            for j in range(nsub):
                pltpu.make_async_copy(rows2.at[b].at[pl.ds(j * _SUB, _SUB)],
                                      acc.at[dst3.at[b].at[j]], sem_sc).wait()

        def multiply(c):
            b = c % 2

            def mg(g, car):
                ew16 = ew2[b, pl.ds(g * _LANES, _LANES)]
                for e in range(_LANES):
                    ea = g * _LANES + e
                    w = _splat(ew16, e)
                    rows2[b, ea, pl.ds(0, _LANES)] = (
                        rows2[b, ea, pl.ds(0, _LANES)] * w)
                return car
            lax.fori_loop(0, _CE // _LANES, mg, 0)

        def sweep(tbl, base0, rbase0, nch_l, off, use_idx):
            stage_issue(0, base0)
            dst_issue(0, rbase0)
            dst_issue(1, rbase0)
            stage_drain(0, base0)
            if use_idx:
                idx_compute(0, off)
            gathers_issue(0, tbl, use_idx)
            stage_issue(1, base0)

            def step(i, car):
                nxt = i + 1

                @pl.when(nxt < nch_l)
                def _pre():
                    stage_drain(nxt, base0)
                    if use_idx:
                        idx_compute(nxt, off)

                    @pl.when(i >= 1)
                    def _ds():
                        scatters_drain(nxt)
                        dst_issue(nxt, rbase0)
                    gathers_issue(nxt, tbl, use_idx)

                gathers_drain(i, tbl, use_idx)
                multiply(i)
                dst_wait(i, rbase0)
                scatters_issue(i)

                @pl.when(i + 2 < nch_l)
                def _st():
                    stage_issue(i + 2, base0)
                return car
            lax.fori_loop(0, nch_l, step, 0)
            scatters_drain(nch_l - 2)
            scatters_drain(nch_l - 1)

        def zero_acc():
            def zc(q, car):
                c = tile + q * _NT

                @pl.when(c < nrch)
                def _():
                    pltpu.sync_copy(zbuf, acc.at[pl.ds(c * _RCH, _RCH)])
                return car
            lax.fori_loop(0, qmax, zc, 0)

        def flush(sidx):
            def fc(q, car):
                c = tile + q * _NT

                @pl.when(c < nrch)
                def _():
                    pltpu.sync_copy(
                        acc.at[pl.ds(c * _RCH, _RCH)],
                        out_hbm.at[pl.ds(c * _RCH, _RCH)]
                               .at[:, pl.ds(sidx * _WS, _WS)])
                return car
            lax.fori_loop(0, qmax, fc, 0)

        for k in range(num_hslabs // 2):
            s = 2 * k + core
            off = jnp.where(s >= 4, s - 4 + 4 * n_nodes, s).astype(jnp.int32)
            zero_acc()
            plsc.subcore_barrier()
            sweep(h_hbm, tile * te, tile * (te // _SUB), nch, off, True)
            plsc.subcore_barrier()
            flush(s)
            plsc.subcore_barrier()

        if has_x:
            base0 = core * (e_pad // 2) + tile * te_x
            rbase0 = core * (e_pad // (2 * _SUB)) + tile * (te_x // _SUB)
            zero_acc()
            plsc.subcore_barrier()
            sweep(x_hbm, base0, rbase0, nch_x, None, False)
            plsc.subcore_barrier()
            flush((num_hslabs + core).astype(jnp.int32))
            plsc.subcore_barrier()

    args = (htable, xtable, src_flat, dst2, ew_flat, zrows) if has_x else (
        htable, src_flat, dst2, ew_flat, zrows)
    return pl.kernel(
        body,
        out_type=jax.ShapeDtypeStruct((n_pad, nout * _WS), F32),
        mesh=mesh,
        compiler_params=pltpu.CompilerParams(use_tc_tiling_on_sc=False),
        scratch_types=[
            pltpu.VMEM((2, _CE), jnp.int32),
            pltpu.VMEM((2, _CE), jnp.int32),
            pltpu.VMEM((2, _CE // _SUB, _SUB), jnp.int32),
            pltpu.VMEM((2, _CE), F32),
            pltpu.VMEM((2, _CE, _WS), F32),
            pltpu.VMEM((_RCH, _WS), F32),
            pltpu.VMEM_SHARED((n_pad, _WS), F32),
            pltpu.SemaphoreType.DMA,
            pltpu.SemaphoreType.DMA,
            pltpu.SemaphoreType.DMA,
            pltpu.SemaphoreType.DMA,
        ],
    )(*args)


def _ln_b(v, g, b):
    mu = jnp.mean(v, axis=-1, keepdims=True)
    var = jnp.mean((v - mu) ** 2, axis=-1, keepdims=True)
    return (v - mu) * lax.rsqrt(var + 1e-5) * g + b


def _tc_layer0(s0, c0, ws, bs, gh, bh, gc, bcc, tn, n):
    kdim = s0.shape[1]
    grid = n // tn

    def body(s_ref, c_ref, wi, wf, wc, wo, bi, bf, bc_, bo,
             gh_r, bh_r, gc_r, bcc_r, hid_o, cel_o):
        s = s_ref[...]
        pi = jnp.dot(s, wi[...], preferred_element_type=F32) + bi[...]
        pf = jnp.dot(s, wf[...], preferred_element_type=F32) + bf[...]
        pc = jnp.dot(s, wc[...], preferred_element_type=F32) + bc_[...]
        po = jnp.dot(s, wo[...], preferred_element_type=F32) + bo[...]
        i_ = jax.nn.sigmoid(pi)
        f_ = jax.nn.sigmoid(pf)
        g_ = jnp.tanh(pc)
        o_ = jax.nn.sigmoid(po)
        cn = f_ * c_ref[...] + i_ * g_
        hn = o_ * jnp.tanh(cn)
        hid_o[...] = _ln_b(hn, gh_r[...], bh_r[...])
        cel_o[...] = _ln_b(cn, gc_r[...], bcc_r[...])

    row = lambda i: (i, 0)
    fix = lambda i: (0, 0)
    return pl.pallas_call(
        body,
        grid=(grid,),
        in_specs=[
            pl.BlockSpec((tn, kdim), row), pl.BlockSpec((tn, 64), row),
        ] + [pl.BlockSpec((kdim, 64), fix)] * 4
          + [pl.BlockSpec((1, 64), fix)] * 8,
        out_specs=[pl.BlockSpec((tn, 64), row)] * 2,
        out_shape=[jax.ShapeDtypeStruct((n, 64), F32)] * 2,
    )(s0, c0, *ws, *bs, gh, bh, gc, bcc)


def _tc_layer1(s1, sb, c1, skip, ws, bs, gh, bh, gc, bcc, go, bo_ln,
               fa, fb, fbias, f2w, f2b, tn, n):
    kdim = s1.shape[1]
    grid = n // tn

    def body(s_ref, sb_ref, c_ref, sk_ref,
             wxi, wxf, wxc, wxo, whi, whf, whc, who, bi, bf, bc_, bo,
             gh_r, bh_r, gc_r, bcc_r, go_r, bol_r,
             fa_r, fb_r, fbias_r, f2w_r, f2b_r,
             hid_o, cel_o, o_out):
        s = s_ref[...]
        sb_ = sb_ref[...]
        pi = (jnp.dot(s, wxi[...], preferred_element_type=F32)
              + jnp.dot(sb_, whi[...], preferred_element_type=F32) + bi[...])
        pf = (jnp.dot(s, wxf[...], preferred_element_type=F32)
              + jnp.dot(sb_, whf[...], preferred_element_type=F32) + bf[...])
        pc = (jnp.dot(s, wxc[...], preferred_element_type=F32)
              + jnp.dot(sb_, whc[...], preferred_element_type=F32) + bc_[...])
        po = (jnp.dot(s, wxo[...], preferred_element_type=F32)
              + jnp.dot(sb_, who[...], preferred_element_type=F32) + bo[...])
        i_ = jax.nn.sigmoid(pi)
        f_ = jax.nn.sigmoid(pf)
        g_ = jnp.tanh(pc)
        o_ = jax.nn.sigmoid(po)
        cn = f_ * c_ref[...] + i_ * g_
        hn = o_ * jnp.tanh(cn)
        hid_o[...] = _ln_b(hn, gh_r[...], bh_r[...])
        cel_o[...] = _ln_b(cn, gc_r[...], bcc_r[...])
        ob = jnp.maximum(_ln_b(hn, go_r[...], bol_r[...]), 0.0)
        t = (jnp.dot(ob, fa_r[...], preferred_element_type=F32)
             + sk_ref[...] * fb_r[...] + fbias_r[...])
        t = jnp.maximum(t, 0.0)
        o_out[...] = jax.nn.sigmoid(
            jnp.sum(t * f2w_r[...], axis=-1, keepdims=True) + f2b_r[...])

    row = lambda i: (i, 0)
    fix = lambda i: (0, 0)
    return pl.pallas_call(
        body,
        grid=(grid,),
        in_specs=[
            pl.BlockSpec((tn, kdim), row), pl.BlockSpec((tn, 64), row),
            pl.BlockSpec((tn, 64), row), pl.BlockSpec((tn, 1), row),
        ] + [pl.BlockSpec((kdim, 64), fix)] * 8
          + [pl.BlockSpec((1, 64), fix)] * 10
          + [pl.BlockSpec((64, 64), fix)]
          + [pl.BlockSpec((1, 64), fix)] * 3
          + [pl.BlockSpec((1, 1), fix)],
        out_specs=[pl.BlockSpec((tn, 64), row)] * 2 + [pl.BlockSpec((tn, 1), row)],
        out_shape=[jax.ShapeDtypeStruct((n, 64), F32)] * 2
                  + [jax.ShapeDtypeStruct((n, 1), F32)],
    )(s1, sb, c1, skip, *ws, *bs, gh, bh, gc, bcc, go, bo_ln,
      fa, fb, fbias, f2w, f2b)


def kernel(x, edge_index, edge_weight, skip, H, C, params):
    n = x.shape[0]
    e = edge_index.shape[1]
    fin = x.shape[1]
    n_pad = ((n + _RCH - 1) // _RCH) * _RCH
    e_blk = 2 * _NT * _CE
    e_pad = ((e + e_blk - 1) // e_blk) * e_blk
    padw = e_pad - e

    src = edge_index[0]
    dst = edge_index[1]
    srcf = jnp.concatenate([src, jnp.zeros((padw,), jnp.int32)])
    dstf = jnp.concatenate([dst, jnp.zeros((padw,), jnp.int32)])
    ewf = jnp.concatenate([edge_weight, jnp.zeros((padw,), F32)])
    dst2 = dstf.reshape(-1, _SUB)
    zrows = jnp.zeros((_RCH, _WS), F32)

    H0r = H[0].reshape(-1, _WS)
    H1r = H[1].reshape(-1, _WS)
    xp = jnp.pad(x, ((0, 0), (0, _WS - fin)))
    S = _sc_spmm(H0r, xp, srcf, dst2, ewf, zrows, 4, n, n_pad)
    Sb = _sc_spmm(H1r, None, srcf, dst2, ewf, zrows, 4, n, n_pad)

    l0, l1 = params['layers'][0], params['layers'][1]
    gates = ('i', 'f', 'c', 'o')
    wxp0 = [jnp.pad(l0['Wx_' + g], ((0, _WS - fin), (0, 0))) for g in gates]
    ws0 = [jnp.concatenate([l0['Wh_' + g], wxp0[j], wxp0[j]], axis=0)
           for j, g in enumerate(gates)]
    bs0 = [l0['b_' + g].reshape(1, 64) for g in gates]
    gh = params['ln_h_g'].reshape(1, 64)
    bh = params['ln_h_b'].reshape(1, 64)
    gc = params['ln_c_g'].reshape(1, 64)
    bcc = params['ln_c_b'].reshape(1, 64)

    tn = 2000
    hid0, cel0 = _tc_layer0(S, C[0], ws0, bs0, gh, bh, gc, bcc, tn, n)

    hid0r = hid0.reshape(-1, _WS)
    S2 = _sc_spmm(hid0r, None, srcf, dst2, ewf, zrows, 4, n, n_pad)

    ws1 = ([l1['Wx_' + g] for g in gates] + [l1['Wh_' + g] for g in gates])
    bs1 = [l1['b_' + g].reshape(1, 64) for g in gates]
    go = params['ln_o_g'].reshape(1, 64)
    bo_ln = params['ln_o_b'].reshape(1, 64)
    fa = params['fc1_W'][:64]
    fb = params['fc1_W'][64:65]
    fbias = params['fc1_b'].reshape(1, 64)
    f2w = params['fc2_W'].T
    f2b = params['fc2_b'].reshape(1, 1)

    hid1, cel1, o = _tc_layer1(S2, Sb, C[1], skip, ws1, bs1, gh, bh, gc, bcc,
                               go, bo_ln, fa, fb, fbias, f2w, f2b, tn, n)

    hidden = jnp.stack([hid0, hid1])
    cell = jnp.stack([cel0, cel1])
    return o, hidden, cell
